# Initial kernel scaffold; baseline (speedup 1.0000x reference)
import jax
import jax.numpy as jnp
from jax.experimental import pallas as pl


def kernel(tensor, emb1_weight, emb2_weight):
    raise NotImplementedError("write your pallas kernel here")


# speedup vs baseline: 2.7441x; 2.7441x over previous
import functools

import jax
import jax.numpy as jnp
from jax import lax
from jax.experimental import pallas as pl
from jax.experimental.pallas import tpu as pltpu
from jax.experimental.pallas import tpu_sc as plsc

DIVIDER = 1000
BATCH = 16384
HIDDEN = 64
LANES = 16
NUM_WORKERS = 32
BPW = BATCH // NUM_WORKERS
CHUNK = 128
NCHUNKS = BPW // CHUNK


_mesh = plsc.VectorSubcoreMesh(core_axis_name="c", subcore_axis_name="s")


@functools.partial(
    pl.kernel,
    mesh=_mesh,
    out_type=jax.ShapeDtypeStruct((BATCH, HIDDEN), jnp.float32),
    scratch_types=[
        pltpu.VMEM((BPW,), jnp.int32),
        pltpu.VMEM((BPW,), jnp.int32),
        pltpu.VMEM((BPW,), jnp.int32),
        pltpu.VMEM((BPW, HIDDEN), jnp.float32),
        pltpu.VMEM((BPW, HIDDEN), jnp.float32),
        pltpu.SemaphoreType.DMA,
    ],
    compiler_params=pltpu.CompilerParams(use_tc_tiling_on_sc=False),
)
def _qr_embed(idx_hbm, emb1_hbm, emb2_hbm, out_hbm,
              idx_v, i1_v, i2_v, rows1_v, rows2_v, sem):
    wid = lax.axis_index("s") * 2 + lax.axis_index("c")
    base = wid * BPW

    pltpu.sync_copy(idx_hbm.at[pl.ds(base, BPW)], idx_v)

    div_vec = jnp.full((LANES,), DIVIDER, dtype=jnp.int32)

    def split_idx(j, carry):
        sl = pl.ds(j * LANES, LANES)
        v = idx_v[sl]
        q = lax.div(v, div_vec)
        i2_v[sl] = q
        i1_v[sl] = lax.sub(v, lax.mul(q, div_vec))
        return carry

    lax.fori_loop(0, BPW // LANES, split_idx, 0)

    copies = []
    for k in range(NCHUNKS):
        row_sl = pl.ds(k * CHUNK, CHUNK)
        copies.append(pltpu.async_copy(
            emb1_hbm.at[i1_v.at[row_sl]], rows1_v.at[row_sl], sem))
        copies.append(pltpu.async_copy(
            emb2_hbm.at[i2_v.at[row_sl]], rows2_v.at[row_sl], sem))
    for c in copies:
        c.wait()

    def mul_row(r, carry):
        for c in range(HIDDEN // LANES):
            sl = pl.ds(c * LANES, LANES)
            rows1_v[r, sl] = rows1_v[r, sl] * rows2_v[r, sl]
        return carry

    lax.fori_loop(0, BPW, mul_row, 0)

    pltpu.sync_copy(rows1_v, out_hbm.at[pl.ds(base, BPW)])


def kernel(tensor, emb1_weight, emb2_weight):
    idx = tensor.astype(jnp.int32)
    return _qr_embed(idx, emb1_weight, emb2_weight)


# speedup vs baseline: 2.7464x; 1.0008x over previous
import functools

import jax
import jax.numpy as jnp
from jax import lax
from jax.experimental import pallas as pl
from jax.experimental.pallas import tpu as pltpu
from jax.experimental.pallas import tpu_sc as plsc

DIVIDER = 1000
BATCH = 16384
HIDDEN = 64
LANES = 16
NUM_WORKERS = 32
BPW = BATCH // NUM_WORKERS
CHUNK = 128
NCHUNKS = BPW // CHUNK
ROW_UNROLL = 4


_mesh = plsc.VectorSubcoreMesh(core_axis_name="c", subcore_axis_name="s")


@functools.partial(
    pl.kernel,
    mesh=_mesh,
    out_type=jax.ShapeDtypeStruct((BATCH, HIDDEN), jnp.float32),
    scratch_types=[
        pltpu.VMEM((BPW,), jnp.int32),
        pltpu.VMEM((BPW,), jnp.int32),
        pltpu.VMEM((BPW,), jnp.int32),
        pltpu.VMEM((BPW, HIDDEN), jnp.float32),
        pltpu.VMEM((BPW, HIDDEN), jnp.float32),
        [pltpu.SemaphoreType.DMA] * NCHUNKS,
        pltpu.SemaphoreType.DMA,
    ],
    compiler_params=pltpu.CompilerParams(use_tc_tiling_on_sc=False),
)
def _qr_embed(idx_hbm, emb1_hbm, emb2_hbm, out_hbm,
              idx_v, i1_v, i2_v, rows1_v, rows2_v, gsems, ssem):
    wid = lax.axis_index("s") * 2 + lax.axis_index("c")
    base = wid * BPW

    pltpu.sync_copy(idx_hbm.at[pl.ds(base, BPW)], idx_v)

    div_vec = jnp.full((LANES,), DIVIDER, dtype=jnp.int32)

    gathers = []
    for k in range(NCHUNKS):
        def split_idx(j, carry, k=k):
            sl = pl.ds(k * CHUNK + j * LANES, LANES)
            v = idx_v[sl]
            q = lax.div(v, div_vec)
            i2_v[sl] = q
            i1_v[sl] = lax.sub(v, lax.mul(q, div_vec))
            return carry

        lax.fori_loop(0, CHUNK // LANES, split_idx, 0)
        row_sl = pl.ds(k * CHUNK, CHUNK)
        gathers.append((
            pltpu.async_copy(emb1_hbm.at[i1_v.at[row_sl]],
                             rows1_v.at[row_sl], gsems[k]),
            pltpu.async_copy(emb2_hbm.at[i2_v.at[row_sl]],
                             rows2_v.at[row_sl], gsems[k]),
        ))

    stores = []
    for k in range(NCHUNKS):
        g1, g2 = gathers[k]
        g1.wait()
        g2.wait()

        def mul_rows(r, carry, k=k):
            row0 = k * CHUNK + r * ROW_UNROLL
            for u in range(ROW_UNROLL):
                for c in range(HIDDEN // LANES):
                    sl = pl.ds(c * LANES, LANES)
                    rows1_v[row0 + u, sl] = (
                        rows1_v[row0 + u, sl] * rows2_v[row0 + u, sl])
            return carry

        lax.fori_loop(0, CHUNK // ROW_UNROLL, mul_rows, 0)
        row_sl = pl.ds(k * CHUNK, CHUNK)
        stores.append(pltpu.async_copy(
            rows1_v.at[row_sl],
            out_hbm.at[pl.ds(base + k * CHUNK, CHUNK)], ssem))

    for s in stores:
        s.wait()


def kernel(tensor, emb1_weight, emb2_weight):
    idx = tensor.astype(jnp.int32)
    return _qr_embed(idx, emb1_weight, emb2_weight)
